# baseline (speedup 1.0000x reference)
import jax
import jax.numpy as jnp
from jax import lax
from jax.experimental import pallas as pl
from jax.experimental.pallas import tpu as pltpu

TA = 1024
TR = 1024


def _matvec_body(ri_ref, w_ref, x_ref, t_ref, weff_ref):
    @pl.when(pl.program_id(1) == 0)
    def _():
        idx = ri_ref[0]
        w = w_ref[0]
        E = w.shape[0]
        eq = (idx[0][:, None] == lax.broadcasted_iota(jnp.int32, (idx.shape[1], E), 1))
        counts = jnp.sum(eq.astype(jnp.float32), axis=0, keepdims=True)
        maskf = (counts > 0.0).astype(jnp.float32)
        weff_ref[...] = jnp.dot(maskf, w, preferred_element_type=jnp.float32)

    t_ref[0] = lax.dot_general(weff_ref[...], x_ref[0], (((1,), (1,)), ((), ())),
                               preferred_element_type=jnp.float32)


def _outer_body(rw_ref, t_ref, y_ref):
    rw_col = jnp.transpose(rw_ref[0])
    y_ref[0] = rw_col * t_ref[0]


def kernel(x, routing_weights, routing_indices, weights):
    B, S, D = x.shape
    E = weights.shape[0]
    rw = routing_weights.reshape(B, 1, S)
    ri = routing_indices.reshape(B, 1, S)
    wrows = jnp.transpose(weights[:, :B, :], (1, 0, 2))

    t = pl.pallas_call(
        _matvec_body,
        grid=(B, S // TA),
        in_specs=[
            pl.BlockSpec((1, 1, S), lambda j, a: (j, 0, 0)),
            pl.BlockSpec((1, E, D), lambda j, a: (j, 0, 0)),
            pl.BlockSpec((1, TA, D), lambda j, a: (j, a, 0)),
        ],
        out_specs=pl.BlockSpec((1, 1, TA), lambda j, a: (j, 0, a)),
        out_shape=jax.ShapeDtypeStruct((B, 1, S), jnp.float32),
        scratch_shapes=[pltpu.VMEM((1, D), jnp.float32)],
    )(ri, wrows, x)

    return pl.pallas_call(
        _outer_body,
        grid=(B, S // TR),
        in_specs=[
            pl.BlockSpec((1, 1, TR), lambda j, r: (j, 0, r)),
            pl.BlockSpec((1, 1, S), lambda j, r: (j, 0, 0)),
        ],
        out_specs=pl.BlockSpec((1, TR, S), lambda j, r: (j, r, 0)),
        out_shape=jax.ShapeDtypeStruct((B, S, S), x.dtype),
    )(rw, t)
